# Initial kernel scaffold; baseline (speedup 1.0000x reference)
import jax
import jax.numpy as jnp
from jax.experimental import pallas as pl


def kernel(xyz, new_xyz, features, fps_idx):
    raise NotImplementedError("write your pallas kernel here")


# speedup vs baseline: 3.3053x; 3.3053x over previous
import functools

import jax
import jax.numpy as jnp
from jax import lax
from jax.experimental import pallas as pl
from jax.experimental.pallas import tpu as pltpu
from jax.experimental.pallas import tpu_sc as plsc

_RADIUS = 0.2
_NSAMPLE = 64

_B, _N, _S = 8, 4096, 1024
_K = _NSAMPLE + 1
_C = 64
_NW = 32
_QPW = (_B * _S) // _NW
_PLANE = _QPW * _K
_LANES = 16


def _select_idx(xyz, new_xyz, fps_idx):
    sqrdists = (jnp.sum(new_xyz ** 2, axis=-1)[:, :, None]
                + jnp.sum(xyz ** 2, axis=-1)[:, None, :]
                - 2.0 * jnp.einsum('bsd,bnd->bsn', new_xyz, xyz))
    arange = jnp.arange(_N, dtype=jnp.int32)[None, None, :]
    group_idx = jnp.where(sqrdists > _RADIUS * _RADIUS, jnp.int32(_N), arange)
    group_idx = -lax.top_k(-group_idx, _NSAMPLE)[0]
    group_first = group_idx[:, :, 0:1]
    group_first = jnp.where(group_first == _N, jnp.int32(0), group_first)
    group_idx = jnp.where(group_idx == _N, group_first, group_idx)
    return jnp.concatenate([fps_idx[:, :, None].astype(jnp.int32), group_idx],
                           axis=2)


def _sc_gather(table, idx_flat, qsub):
    mesh = plsc.VectorSubcoreMesh(core_axis_name="c", subcore_axis_name="s")

    @functools.partial(
        pl.kernel,
        out_type=jax.ShapeDtypeStruct((_B * 70 * _S * _K,), jnp.float32),
        mesh=mesh,
        scratch_types=[
            pltpu.VMEM((_N,), jnp.float32),
            pltpu.VMEM((_PLANE,), jnp.int32),
            pltpu.VMEM((_PLANE,), jnp.float32),
            pltpu.VMEM((_PLANE,), jnp.float32),
        ],
        compiler_params=pltpu.CompilerParams(needs_layout_passes=False),
    )
    def k(table_hbm, idx_hbm, qsub_hbm, out_hbm, frow, idxv, qv, outv):
        wid = lax.axis_index("s") * 2 + lax.axis_index("c")
        b = wid // 4
        e0 = (wid % 4) * _PLANE
        spk = _S * _K

        pltpu.sync_copy(idx_hbm.at[pl.ds(b * spk + e0, _PLANE)], idxv)

        def gather_plane(sub):
            def body(t, _):
                sl = pl.ds(t * _LANES, _LANES)
                vals = plsc.load_gather(frow, [idxv[sl]])
                if sub:
                    vals = vals - qv[sl]
                outv[sl] = vals
                return 0
            lax.fori_loop(0, _PLANE // _LANES, body, 0)

        def out_off(c):
            return (b * 70 + c) * spk + e0

        for comp in range(3):
            pltpu.sync_copy(table_hbm.at[pl.ds((b * 67 + comp) * _N, _N)], frow)
            pltpu.sync_copy(
                qsub_hbm.at[pl.ds((b * 3 + comp) * spk + e0, _PLANE)], qv)
            gather_plane(sub=True)
            pltpu.sync_copy(outv, out_hbm.at[pl.ds(out_off(comp), _PLANE)])
            pltpu.sync_copy(outv, out_hbm.at[pl.ds(out_off(comp + 3), _PLANE)])

        for c in range(_C):
            pltpu.sync_copy(table_hbm.at[pl.ds((b * 67 + 3 + c) * _N, _N)], frow)
            gather_plane(sub=False)
            pltpu.sync_copy(outv, out_hbm.at[pl.ds(out_off(6 + c), _PLANE)])

    return k(table, idx_flat, qsub)


def kernel(xyz, new_xyz, features, fps_idx):
    idx = _select_idx(xyz, new_xyz, fps_idx)
    table = jnp.concatenate(
        [jnp.transpose(xyz, (0, 2, 1)), features], axis=1)
    idx_flat = idx.reshape(-1)
    qsub = jnp.broadcast_to(
        jnp.transpose(new_xyz, (0, 2, 1))[:, :, :, None],
        (_B, 3, _S, _K)).reshape(-1)
    out = _sc_gather(table.reshape(-1), idx_flat, qsub)
    return out.reshape(_B, 70, _S, _K)


# speedup vs baseline: 16.9194x; 5.1189x over previous
import functools

import jax
import jax.numpy as jnp
from jax import lax
from jax.experimental import pallas as pl
from jax.experimental.pallas import tpu as pltpu
from jax.experimental.pallas import tpu_sc as plsc

_RADIUS = 0.2
_NSAMPLE = 64

_B, _N, _S = 8, 4096, 1024
_K = _NSAMPLE + 1
_C = 64
_NW = 32
_QPW = (_B * _S) // _NW
_PLANE = _QPW * _K
_LANES = 16
_NWORD = _N // 32


def _mask_words(xyz, new_xyz):
    sqrdists = (jnp.sum(new_xyz ** 2, axis=-1)[:, :, None]
                + jnp.sum(xyz ** 2, axis=-1)[:, None, :]
                - 2.0 * jnp.einsum('bsd,bnd->bsn', new_xyz, xyz))
    inball = ~(sqrdists > _RADIUS * _RADIUS)
    bits = inball.reshape(_B, _S, _NWORD, 32).astype(jnp.uint32)
    weights = jnp.uint32(1) << jnp.arange(32, dtype=jnp.uint32)
    words = (bits * weights).sum(-1, dtype=jnp.uint32)
    return lax.bitcast_convert_type(words, jnp.int32)


def _sc_kernel(table, words, fps, qsub):
    mesh = plsc.VectorSubcoreMesh(core_axis_name="c", subcore_axis_name="s")

    @functools.partial(
        pl.kernel,
        out_type=jax.ShapeDtypeStruct((_B * 70 * _S * _K,), jnp.float32),
        mesh=mesh,
        scratch_types=[
            pltpu.VMEM((_N,), jnp.float32),
            pltpu.VMEM((_QPW * _NWORD,), jnp.int32),
            pltpu.VMEM((_QPW,), jnp.int32),
            pltpu.VMEM((_PLANE,), jnp.int32),
            pltpu.VMEM((_NSAMPLE,), jnp.int32),
            pltpu.VMEM((_PLANE,), jnp.float32),
            pltpu.VMEM((_PLANE,), jnp.float32),
        ],
        compiler_params=pltpu.CompilerParams(needs_layout_passes=False),
    )
    def k(table_hbm, words_hbm, fps_hbm, qsub_hbm, out_hbm,
          frow, wordsv, fpsv, idxv, tempv, qv, outv):
        wid = lax.axis_index("s") * 2 + lax.axis_index("c")
        b = wid // 4
        e0 = (wid % 4) * _PLANE
        q0 = wid * _QPW
        spk = _S * _K

        pltpu.sync_copy(words_hbm.at[pl.ds(q0 * _NWORD, _QPW * _NWORD)],
                        wordsv)
        pltpu.sync_copy(fps_hbm.at[pl.ds(q0, _QPW)], fpsv)

        iota = lax.iota(jnp.int32, _LANES)
        ones = jnp.ones((_LANES,), jnp.int32)

        def fps_body(t, _):
            vals = fpsv[pl.ds(t * _LANES, _LANES)]
            positions = (jnp.full((_LANES,), t * _LANES, jnp.int32)
                         + iota) * _K
            plsc.store_scatter(idxv, [positions], vals)
            return 0

        lax.fori_loop(0, _QPW // _LANES, fps_body, 0)

        def select_query(q, _):
            wbase = q * _NWORD

            def half(wv, sh, base_vals, pos_v):
                bitv = lax.shift_right_logical(wv, sh) & 1
                m = bitv == 1
                positions = pos_v + plsc.cumsum(bitv) - 1
                wmask = m & (positions <= _NSAMPLE - 1)
                plsc.store_scatter(tempv, [positions], base_vals, mask=wmask)
                return pos_v + plsc.all_reduce_population_count(m)

            def chunk_body(carry):
                w, pos_v = carry
                wvec = wordsv[pl.ds(wbase + w, _LANES)]
                for i in range(_LANES):
                    wv = jnp.full((_LANES,), wvec[i], jnp.int32)
                    base = jnp.full((_LANES,), (w + i) * 32, jnp.int32) + iota
                    pos_v = half(wv, iota, base, pos_v)
                    pos_v = half(wv, iota + 16, base + 16, pos_v)
                return w + _LANES, pos_v

            def chunk_cond(carry):
                w, pos_v = carry
                return (w < _NWORD) & (jnp.max(pos_v) < _NSAMPLE)

            _, pos_v = lax.while_loop(
                chunk_cond, chunk_body, (0, jnp.zeros((_LANES,), jnp.int32)))

            pos = jnp.max(pos_v)
            t0 = tempv[pl.ds(0, _LANES)]
            first = jnp.where(pos > 0, t0[0], 0)
            fillv = jnp.full((_LANES,), first, jnp.int32)
            obase = q * _K
            for j in range(_NSAMPLE // _LANES):
                v = tempv[pl.ds(j * _LANES, _LANES)]
                m = (iota + j * _LANES) < pos_v
                idxv[pl.ds(obase + 1 + j * _LANES, _LANES)] = (
                    jnp.where(m, v, fillv))
            return 0

        lax.fori_loop(0, _QPW, select_query, 0)

        def gather_plane(sub):
            def body(t, _):
                sl = pl.ds(t * _LANES, _LANES)
                vals = plsc.load_gather(frow, [idxv[sl]])
                if sub:
                    vals = vals - qv[sl]
                outv[sl] = vals
                return 0
            lax.fori_loop(0, _PLANE // _LANES, body, 0)

        def out_off(c):
            return (b * 70 + c) * spk + e0

        for comp in range(3):
            pltpu.sync_copy(table_hbm.at[pl.ds((b * 67 + comp) * _N, _N)], frow)
            pltpu.sync_copy(
                qsub_hbm.at[pl.ds((b * 3 + comp) * spk + e0, _PLANE)], qv)
            gather_plane(sub=True)
            pltpu.sync_copy(outv, out_hbm.at[pl.ds(out_off(comp), _PLANE)])
            pltpu.sync_copy(outv, out_hbm.at[pl.ds(out_off(comp + 3), _PLANE)])

        for c in range(_C):
            pltpu.sync_copy(table_hbm.at[pl.ds((b * 67 + 3 + c) * _N, _N)], frow)
            gather_plane(sub=False)
            pltpu.sync_copy(outv, out_hbm.at[pl.ds(out_off(6 + c), _PLANE)])

    return k(table, words, fps, qsub)


def kernel(xyz, new_xyz, features, fps_idx):
    words = _mask_words(xyz, new_xyz)
    table = jnp.concatenate(
        [jnp.transpose(xyz, (0, 2, 1)), features], axis=1)
    qsub = jnp.broadcast_to(
        jnp.transpose(new_xyz, (0, 2, 1))[:, :, :, None],
        (_B, 3, _S, _K)).reshape(-1)
    out = _sc_kernel(table.reshape(-1), words.reshape(-1),
                     fps_idx.astype(jnp.int32).reshape(-1), qsub)
    return out.reshape(_B, 70, _S, _K)


# speedup vs baseline: 17.9393x; 1.0603x over previous
import functools

import jax
import jax.numpy as jnp
import numpy as np
from jax import lax
from jax.experimental import pallas as pl
from jax.experimental.pallas import tpu as pltpu
from jax.experimental.pallas import tpu_sc as plsc

_RADIUS = 0.2
_NSAMPLE = 64

_B, _N, _S = 8, 4096, 1024
_K = _NSAMPLE + 1
_C = 64
_NW = 32
_QPW = (_B * _S) // _NW
_PLANE = _QPW * _K
_LANES = 16
_NWORD = _N // 32
_UNROLL = 4


def _mask_words(xyz, new_xyz):
    sqrdists = (jnp.sum(new_xyz ** 2, axis=-1)[:, :, None]
                + jnp.sum(xyz ** 2, axis=-1)[:, None, :]
                - 2.0 * jnp.einsum('bsd,bnd->bsn', new_xyz, xyz))
    inball = ~(sqrdists > _RADIUS * _RADIUS)
    bits = inball.reshape(_B, _S, _NWORD, 32).astype(jnp.uint32)
    weights = jnp.uint32(1) << jnp.arange(32, dtype=jnp.uint32)
    words = (bits * weights).sum(-1, dtype=jnp.uint32)
    return lax.bitcast_convert_type(words, jnp.int32)


def _sc_kernel(xyzt, feats, words, fps, nxt, qmap):
    mesh = plsc.VectorSubcoreMesh(core_axis_name="c", subcore_axis_name="s")

    @functools.partial(
        pl.kernel,
        out_type=jax.ShapeDtypeStruct((_B * 70 * _S * _K,), jnp.float32),
        mesh=mesh,
        scratch_types=[
            pltpu.VMEM((2 * _N,), jnp.float32),
            pltpu.VMEM((_QPW * _NWORD,), jnp.int32),
            pltpu.VMEM((_QPW,), jnp.int32),
            pltpu.VMEM((_PLANE,), jnp.int32),
            pltpu.VMEM((_NSAMPLE,), jnp.int32),
            pltpu.VMEM((_PLANE,), jnp.int32),
            pltpu.VMEM((3 * _QPW,), jnp.float32),
            pltpu.VMEM((2 * _PLANE,), jnp.float32),
            pltpu.SemaphoreType.DMA,
            pltpu.SemaphoreType.DMA,
            pltpu.SemaphoreType.DMA,
            pltpu.SemaphoreType.DMA,
        ],
        compiler_params=pltpu.CompilerParams(needs_layout_passes=False),
    )
    def k(xyzt_hbm, feats_hbm, words_hbm, fps_hbm, nxt_hbm, qmap_hbm, out_hbm,
          frow, wordsv, fpsv, idxv, tempv, qmapv, nxv, outv,
          fsem0, fsem1, osem0, osem1):
        wid = lax.axis_index("s") * 2 + lax.axis_index("c")
        b = wid // 4
        e0 = (wid % 4) * _PLANE
        q0 = wid * _QPW
        s0 = (wid % 4) * _QPW
        spk = _S * _K

        pltpu.sync_copy(words_hbm.at[pl.ds(q0 * _NWORD, _QPW * _NWORD)],
                        wordsv)
        pltpu.sync_copy(fps_hbm.at[pl.ds(q0, _QPW)], fpsv)
        pltpu.sync_copy(qmap_hbm, qmapv)
        for comp in range(3):
            pltpu.sync_copy(
                nxt_hbm.at[pl.ds((b * 3 + comp) * _S + s0, _QPW)],
                nxv.at[pl.ds(comp * _QPW, _QPW)])

        iota = lax.iota(jnp.int32, _LANES)

        def fps_body(t, _):
            vals = fpsv[pl.ds(t * _LANES, _LANES)]
            positions = (jnp.full((_LANES,), t * _LANES, jnp.int32)
                         + iota) * _K
            plsc.store_scatter(idxv, [positions], vals)
            return 0

        lax.fori_loop(0, _QPW // _LANES, fps_body, 0)

        def select_query(q, _):
            wbase = q * _NWORD

            def half(wv, sh, base_vals, pos_v):
                bitv = lax.shift_right_logical(wv, sh) & 1
                m = bitv == 1
                positions = pos_v + plsc.cumsum(bitv) - 1
                wmask = m & (positions <= _NSAMPLE - 1)
                plsc.store_scatter(tempv, [positions], base_vals, mask=wmask)
                return pos_v + plsc.all_reduce_population_count(m)

            def chunk_body(carry):
                w, pos_v = carry
                wvec = wordsv[pl.ds(wbase + w, _LANES)]
                for i in range(_LANES):
                    wv = jnp.full((_LANES,), wvec[i], jnp.int32)
                    base = jnp.full((_LANES,), (w + i) * 32, jnp.int32) + iota
                    pos_v = half(wv, iota, base, pos_v)
                    pos_v = half(wv, iota + 16, base + 16, pos_v)
                return w + _LANES, pos_v

            def chunk_cond(carry):
                w, pos_v = carry
                return (w < _NWORD) & (jnp.max(pos_v) < _NSAMPLE)

            _, pos_v = lax.while_loop(
                chunk_cond, chunk_body, (0, jnp.zeros((_LANES,), jnp.int32)))

            pos = jnp.max(pos_v)
            t0 = tempv[pl.ds(0, _LANES)]
            first = jnp.where(pos > 0, t0[0], 0)
            fillv = jnp.full((_LANES,), first, jnp.int32)
            obase = q * _K
            for j in range(_NSAMPLE // _LANES):
                v = tempv[pl.ds(j * _LANES, _LANES)]
                m = (iota + j * _LANES) < pos_v
                idxv[pl.ds(obase + 1 + j * _LANES, _LANES)] = (
                    jnp.where(m, v, fillv))
            return 0

        lax.fori_loop(0, _QPW, select_query, 0)

        sched = ([((b * 3 + comp) * _N, comp, [comp, comp + 3])
                  for comp in range(3)]
                 + [((b * _C + c) * _N, None, [6 + c]) for c in range(_C)])
        fsems = [fsem0, fsem1]
        osems = [osem0, osem1]
        out_handles = {0: [], 1: []}

        def row_src(i):
            off, _, _ = sched[i]
            src = xyzt_hbm if i < 3 else feats_hbm
            return src.at[pl.ds(off, _N)]

        def gather_plane(par, comp):
            def body(t, _):
                for u in range(_UNROLL):
                    j = (t * _UNROLL + u) * _LANES
                    sl = pl.ds(j, _LANES)
                    vals = plsc.load_gather(frow.at[pl.ds(par * _N, _N)], [idxv[sl]])
                    if comp is not None:
                        qvals = plsc.load_gather(nxv.at[pl.ds(comp * _QPW, _QPW)], [qmapv[sl]])
                        vals = vals - qvals
                    outv[pl.ds(par * _PLANE + j, _LANES)] = vals
                return 0
            lax.fori_loop(0, _PLANE // (_LANES * _UNROLL), body, 0)

        h = pltpu.async_copy(row_src(0), frow.at[pl.ds(0, _N)], fsems[0])
        for i, (off, comp, outs) in enumerate(sched):
            par = i % 2
            h.wait()
            if i + 1 < len(sched):
                h = pltpu.async_copy(row_src(i + 1), frow.at[pl.ds((1 - par) * _N, _N)],
                                     fsems[1 - par])
            for oh in out_handles[par]:
                oh.wait()
            out_handles[par] = []
            gather_plane(par, comp)
            for c_out in outs:
                out_handles[par].append(pltpu.async_copy(
                    outv.at[pl.ds(par * _PLANE, _PLANE)],
                    out_hbm.at[pl.ds((b * 70 + c_out) * spk + e0, _PLANE)],
                    osems[par]))
        for par in range(2):
            for oh in out_handles[par]:
                oh.wait()

    return k(xyzt, feats, words, fps, nxt, qmap)


def kernel(xyz, new_xyz, features, fps_idx):
    words = _mask_words(xyz, new_xyz)
    xyzt = jnp.transpose(xyz, (0, 2, 1)).reshape(-1)
    nxt = jnp.transpose(new_xyz, (0, 2, 1)).reshape(-1)
    qmap = jnp.asarray(
        np.repeat(np.arange(_QPW, dtype=np.int32), _K))
    out = _sc_kernel(xyzt, features.reshape(-1), words.reshape(-1),
                     fps_idx.astype(jnp.int32).reshape(-1), nxt, qmap)
    return out.reshape(_B, 70, _S, _K)


# speedup vs baseline: 21.3413x; 1.1896x over previous
import functools

import jax
import jax.numpy as jnp
import numpy as np
from jax import lax
from jax.experimental import pallas as pl
from jax.experimental.pallas import tpu as pltpu
from jax.experimental.pallas import tpu_sc as plsc

_RADIUS = 0.2
_NSAMPLE = 64

_B, _N, _S = 8, 4096, 1024
_K = _NSAMPLE + 1
_C = 64
_NW = 32
_QPW = (_B * _S) // _NW
_PLANE = _QPW * _K
_LANES = 16
_NWORD = _N // 32
_UNROLL = 8


_SBLK = 128


def _dist_pack_body(nx3, xt3, qq, pp, out):
    qqv = qq[0, 0]
    ppv = pp[0, 0]
    q3 = nx3[0]
    p3 = xt3[0]
    dot = lax.dot_general(q3, p3, (((1,), (0,)), ((), ())))
    sqrdists = (qqv[:, None] + ppv[None, :]) - 2.0 * dot
    bits = jnp.where(sqrdists > _RADIUS * _RADIUS, 0.0, 1.0)
    n_ids = lax.broadcasted_iota(jnp.int32, (_N, _NWORD), 0)
    w_ids = lax.broadcasted_iota(jnp.int32, (_N, _NWORD), 1)
    in_word = (n_ids // 32) == w_ids
    weight = (1 << (n_ids % 16)).astype(jnp.float32)
    wlo = jnp.where(in_word & ((n_ids % 32) < 16), weight, 0.0)
    whi = jnp.where(in_word & ((n_ids % 32) >= 16), weight, 0.0)
    lo = jax.lax.dot(bits, wlo).astype(jnp.int32)
    hi = jax.lax.dot(bits, whi).astype(jnp.int32)
    out[0] = lo | (hi << 16)


def _mask_words(xyz, new_xyz):
    qq = jnp.sum(new_xyz ** 2, axis=-1)
    pp = jnp.sum(xyz ** 2, axis=-1)
    xt3 = jnp.transpose(xyz, (0, 2, 1))

    grid = (_B, _S // _SBLK)
    return pl.pallas_call(
        _dist_pack_body,
        grid=grid,
        in_specs=[
            pl.BlockSpec((1, _SBLK, 3), lambda b, s: (b, s, 0)),
            pl.BlockSpec((1, 3, _N), lambda b, s: (b, 0, 0)),
            pl.BlockSpec((1, 1, _SBLK), lambda b, s: (b, 0, s)),
            pl.BlockSpec((1, 1, _N), lambda b, s: (b, 0, 0)),
        ],
        out_specs=pl.BlockSpec((1, _SBLK, _NWORD), lambda b, s: (b, s, 0)),
        out_shape=jax.ShapeDtypeStruct((_B, _S, _NWORD), jnp.int32),
    )(new_xyz, xt3, qq.reshape(_B, 1, _S), pp.reshape(_B, 1, _N))


def _sc_kernel(xyzt, feats, words, fps, nxt, qmap):
    mesh = plsc.VectorSubcoreMesh(core_axis_name="c", subcore_axis_name="s")

    @functools.partial(
        pl.kernel,
        out_type=jax.ShapeDtypeStruct((_B * 70 * _S * _K,), jnp.float32),
        mesh=mesh,
        scratch_types=[
            pltpu.VMEM((2 * _N,), jnp.float32),
            pltpu.VMEM((_QPW * _NWORD,), jnp.int32),
            pltpu.VMEM((_QPW,), jnp.int32),
            pltpu.VMEM((_PLANE,), jnp.int32),
            pltpu.VMEM((_NSAMPLE + 512,), jnp.int32),
            pltpu.VMEM((_PLANE,), jnp.int32),
            pltpu.VMEM((3 * _QPW,), jnp.float32),
            pltpu.VMEM((2 * _PLANE,), jnp.float32),
            pltpu.SemaphoreType.DMA,
            pltpu.SemaphoreType.DMA,
            pltpu.SemaphoreType.DMA,
            pltpu.SemaphoreType.DMA,
        ],
        compiler_params=pltpu.CompilerParams(needs_layout_passes=False),
    )
    def k(xyzt_hbm, feats_hbm, words_hbm, fps_hbm, nxt_hbm, qmap_hbm, out_hbm,
          frow, wordsv, fpsv, idxv, tempv, qmapv, nxv, outv,
          fsem0, fsem1, osem0, osem1):
        wid = lax.axis_index("s") * 2 + lax.axis_index("c")
        b = wid // 4
        e0 = (wid % 4) * _PLANE
        q0 = wid * _QPW
        s0 = (wid % 4) * _QPW
        spk = _S * _K

        pltpu.sync_copy(words_hbm.at[pl.ds(q0 * _NWORD, _QPW * _NWORD)],
                        wordsv)
        pltpu.sync_copy(fps_hbm.at[pl.ds(q0, _QPW)], fpsv)
        pltpu.sync_copy(qmap_hbm, qmapv)
        for comp in range(3):
            pltpu.sync_copy(
                nxt_hbm.at[pl.ds((b * 3 + comp) * _S + s0, _QPW)],
                nxv.at[pl.ds(comp * _QPW, _QPW)])

        iota = lax.iota(jnp.int32, _LANES)

        def fps_body(t, _):
            vals = fpsv[pl.ds(t * _LANES, _LANES)]
            positions = (jnp.full((_LANES,), t * _LANES, jnp.int32)
                         + iota) * _K
            plsc.store_scatter(idxv, [positions], vals)
            return 0

        lax.fori_loop(0, _QPW // _LANES, fps_body, 0)

        def select_query(q, _):
            wbase = q * _NWORD

            def half(wv, sh, base_vals, pos_v):
                bitv = lax.shift_right_logical(wv, sh) & 1
                m = bitv == 1
                positions = pos_v + plsc.cumsum(bitv) - 1
                plsc.store_scatter(tempv, [positions], base_vals, mask=m)
                return pos_v + plsc.all_reduce_population_count(m)

            def chunk_body(carry):
                w, pos_v = carry
                wvec = wordsv[pl.ds(wbase + w, _LANES)]
                for i in range(_LANES):
                    wv = jnp.full((_LANES,), wvec[i], jnp.int32)
                    base = jnp.full((_LANES,), (w + i) * 32, jnp.int32) + iota
                    pos_v = half(wv, iota, base, pos_v)
                    pos_v = half(wv, iota + 16, base + 16, pos_v)
                return w + _LANES, pos_v

            def chunk_cond(carry):
                w, pos_v = carry
                return (w < _NWORD) & (jnp.max(pos_v) < _NSAMPLE)

            _, pos_v = lax.while_loop(
                chunk_cond, chunk_body, (0, jnp.zeros((_LANES,), jnp.int32)))

            pos = jnp.max(pos_v)
            t0 = tempv[pl.ds(0, _LANES)]
            first = jnp.where(pos > 0, t0[0], 0)
            fillv = jnp.full((_LANES,), first, jnp.int32)
            obase = q * _K
            for j in range(_NSAMPLE // _LANES):
                v = tempv[pl.ds(j * _LANES, _LANES)]
                m = (iota + j * _LANES) < pos_v
                idxv[pl.ds(obase + 1 + j * _LANES, _LANES)] = (
                    jnp.where(m, v, fillv))
            return 0

        lax.fori_loop(0, _QPW, select_query, 0)

        sched = ([((b * 3 + comp) * _N, comp, [comp, comp + 3])
                  for comp in range(3)]
                 + [((b * _C + c) * _N, None, [6 + c]) for c in range(_C)])
        fsems = [fsem0, fsem1]
        osems = [osem0, osem1]
        out_handles = {0: [], 1: []}

        def row_src(i):
            off, _, _ = sched[i]
            src = xyzt_hbm if i < 3 else feats_hbm
            return src.at[pl.ds(off, _N)]

        def gather_plane(par, comp):
            def body(t, _):
                for u in range(_UNROLL):
                    j = (t * _UNROLL + u) * _LANES
                    sl = pl.ds(j, _LANES)
                    vals = plsc.load_gather(frow.at[pl.ds(par * _N, _N)], [idxv[sl]])
                    if comp is not None:
                        qvals = plsc.load_gather(nxv.at[pl.ds(comp * _QPW, _QPW)], [qmapv[sl]])
                        vals = vals - qvals
                    outv[pl.ds(par * _PLANE + j, _LANES)] = vals
                return 0
            lax.fori_loop(0, _PLANE // (_LANES * _UNROLL), body, 0)

        h = pltpu.async_copy(row_src(0), frow.at[pl.ds(0, _N)], fsems[0])
        for i, (off, comp, outs) in enumerate(sched):
            par = i % 2
            h.wait()
            if i + 1 < len(sched):
                h = pltpu.async_copy(row_src(i + 1), frow.at[pl.ds((1 - par) * _N, _N)],
                                     fsems[1 - par])
            for oh in out_handles[par]:
                oh.wait()
            out_handles[par] = []
            gather_plane(par, comp)
            for c_out in outs:
                out_handles[par].append(pltpu.async_copy(
                    outv.at[pl.ds(par * _PLANE, _PLANE)],
                    out_hbm.at[pl.ds((b * 70 + c_out) * spk + e0, _PLANE)],
                    osems[par]))
        for par in range(2):
            for oh in out_handles[par]:
                oh.wait()

    return k(xyzt, feats, words, fps, nxt, qmap)


def kernel(xyz, new_xyz, features, fps_idx):
    words = _mask_words(xyz, new_xyz)
    xyzt = jnp.transpose(xyz, (0, 2, 1)).reshape(-1)
    nxt = jnp.transpose(new_xyz, (0, 2, 1)).reshape(-1)
    qmap = jnp.asarray(
        np.repeat(np.arange(_QPW, dtype=np.int32), _K))
    out = _sc_kernel(xyzt, features.reshape(-1), words.reshape(-1),
                     fps_idx.astype(jnp.int32).reshape(-1), nxt, qmap)
    return out.reshape(_B, 70, _S, _K)
